# baseline (speedup 1.0000x reference)
import jax
import jax.numpy as jnp
from jax import lax
from jax.experimental import pallas as pl
from jax.experimental.pallas import tpu as pltpu
from jax.experimental.pallas import tpu_sc as plsc

NFEAT = 26
BATCH = 16384
DIM = 32
VOCAB = 100000
NC = 2
NS = 16
NW = NC * NS
BPW = BATCH // NW
FPC = NFEAT // NC

CH = 160
LCH = CH // 4
NCHUNK = VOCAB // CH
NFULL = 39
NRB = 3

NBUF = 4


def _repack_body(*refs):
    tab_refs = refs[:NFEAT]
    l_ref = refs[NFEAT]
    rest = refs[NFEAT + 1:]
    vins = rest[0:NRB]
    vouts = rest[NRB:2 * NRB]
    isems = rest[2 * NRB:3 * NRB]
    osems = rest[3 * NRB:4 * NRB]

    c = lax.axis_index("c")
    tid = lax.axis_index("s")

    def in_copy(tab, ch, b):
        return pltpu.make_async_copy(
            tab.at[pl.ds(ch * CH, CH)], vins[b], isems[b])

    def out_copy(f, ch, b):
        return pltpu.make_async_copy(
            vouts[b], l_ref.at[pl.ds(f * (VOCAB // 4) + ch * LCH, LCH)],
            osems[b])

    def repack(b):
        vin = vins[b]
        vout = vouts[b]

        def q_body(q, carry):
            for m in range(4):
                r = 4 * q + m
                for j in range(2):
                    vout[q, pl.ds(32 * m + 16 * j, 16)] = vin[r, pl.ds(16 * j, 16)]
            return carry

        lax.fori_loop(0, LCH, q_body, 0)

    def conv_table(f):
        tab = tab_refs[f]
        for b in range(NRB):
            in_copy(tab, tid + 16 * b, b).start()

        def step(cc, b):
            ch = tid + 16 * cc
            in_copy(tab, ch, b).wait()

            @pl.when(cc >= NRB)
            def _():
                out_copy(f, tid + 16 * (cc - NRB), b).wait()

            repack(b)
            out_copy(f, ch, b).start()

            @pl.when(cc + NRB < NFULL)
            def _():
                in_copy(tab, ch + 16 * NRB, b).start()

        def body(c4, carry):
            for b in range(NRB):
                step(NRB * c4 + b, b)
            return carry

        lax.fori_loop(0, NFULL // NRB, body, 0)
        for cc in range(NFULL - NRB, NFULL):
            out_copy(f, tid + 16 * cc, cc % NRB).wait()

        @pl.when(tid == 0)
        def _():
            in_copy(tab, NCHUNK - 1, 0).start()
            in_copy(tab, NCHUNK - 1, 0).wait()
            repack(0)
            out_copy(f, NCHUNK - 1, 0).start()
            out_copy(f, NCHUNK - 1, 0).wait()

    @pl.when(c == 0)
    def _():
        for j in range(FPC):
            conv_table(j)

    @pl.when(c == 1)
    def _():
        for j in range(FPC):
            conv_table(FPC + j)


def _gather_body(*refs):
    idx_refs = refs[:NFEAT]
    l_ref = refs[NFEAT]
    out_ref = refs[NFEAT + 1]
    rest = refs[NFEAT + 2:]
    idx_all = rest[0]
    bufs = rest[1:1 + NBUF]
    gsems = rest[1 + NBUF:1 + 2 * NBUF]
    wsems = rest[1 + 2 * NBUF:1 + 3 * NBUF]
    isem = rest[1 + 3 * NBUF]
    wid = lax.axis_index("s") * NC + lax.axis_index("c")
    base = wid * BPW

    ih = [pltpu.async_copy(idx_refs[f].at[pl.ds(base, BPW)], idx_all.at[f], isem)
          for f in range(NFEAT)]
    for h in ih:
        h.wait()

    hg = [None] * NBUF
    hw = [None] * NBUF
    for f in range(NFEAT):
        s = f % NBUF
        if f >= NBUF:
            hw[s].wait()
        hg[s] = pltpu.async_copy(l_ref.at[idx_all.at[f]], bufs[s], gsems[s])
        if f >= NBUF - 1:
            fp = f - (NBUF - 1)
            sp = fp % NBUF
            hg[sp].wait()
            hw[sp] = pltpu.async_copy(
                bufs[sp], out_ref.at[pl.ds(base, BPW), pl.ds(fp * DIM, DIM)],
                wsems[sp])
    for fp in range(NFEAT - (NBUF - 1), NFEAT):
        sp = fp % NBUF
        hg[sp].wait()
        hw[sp] = pltpu.async_copy(
            bufs[sp], out_ref.at[pl.ds(base, BPW), pl.ds(fp * DIM, DIM)],
            wsems[sp])
    for sp in set(fp % NBUF for fp in range(NFEAT - NBUF, NFEAT)):
        hw[sp].wait()


def kernel(f00, f01, f02, f03, f04, f05, f06, f07, f08, f09, f10, f11, f12, f13, f14, f15, f16, f17, f18, f19, f20, f21, f22, f23, f24, f25, W_f00, W_f01, W_f02, W_f03, W_f04, W_f05, W_f06, W_f07, W_f08, W_f09, W_f10, W_f11, W_f12, W_f13, W_f14, W_f15, W_f16, W_f17, W_f18, W_f19, W_f20, W_f21, W_f22, W_f23, W_f24, W_f25):
    raw_idx = (f00, f01, f02, f03, f04, f05, f06, f07, f08, f09, f10, f11,
               f12, f13, f14, f15, f16, f17, f18, f19, f20, f21, f22, f23,
               f24, f25)
    idxs = [jnp.asarray(x, jnp.int32) + jnp.int32(f * VOCAB)
            for f, x in enumerate(raw_idx)]
    tabs = [W_f00, W_f01, W_f02, W_f03, W_f04, W_f05, W_f06, W_f07, W_f08,
            W_f09, W_f10, W_f11, W_f12, W_f13, W_f14, W_f15, W_f16, W_f17,
            W_f18, W_f19, W_f20, W_f21, W_f22, W_f23, W_f24, W_f25]
    mesh = plsc.VectorSubcoreMesh(
        core_axis_name="c", subcore_axis_name="s", num_cores=NC, num_subcores=NS)

    repack = pl.kernel(
        _repack_body,
        out_type=jax.ShapeDtypeStruct((NFEAT * VOCAB // 4, 128), jnp.float32),
        mesh=mesh,
        compiler_params=pltpu.CompilerParams(use_tc_tiling_on_sc=True),
        scratch_types=(
            [pltpu.VMEM((CH, DIM), jnp.float32) for _ in range(NRB)]
            + [pltpu.VMEM((LCH, 128), jnp.float32) for _ in range(NRB)]
            + [pltpu.SemaphoreType.DMA for _ in range(2 * NRB)]
        ),
    )
    l_packed = repack(*tabs)
    l_flat = l_packed.reshape(NFEAT * VOCAB, DIM)

    gather = pl.kernel(
        _gather_body,
        out_type=jax.ShapeDtypeStruct((BATCH, NFEAT * DIM), jnp.float32),
        mesh=mesh,
        compiler_params=pltpu.CompilerParams(use_tc_tiling_on_sc=False),
        scratch_types=(
            [pltpu.VMEM((NFEAT, BPW), jnp.int32)]
            + [pltpu.VMEM((BPW, DIM), jnp.float32) for _ in range(NBUF)]
            + [pltpu.SemaphoreType.DMA for _ in range(2 * NBUF + 1)]
        ),
    )
    return gather(*idxs, l_flat)


# speedup vs baseline: 1.3375x; 1.3375x over previous
import jax
import jax.numpy as jnp
from jax import lax
from jax.experimental import pallas as pl
from jax.experimental.pallas import tpu as pltpu
from jax.experimental.pallas import tpu_sc as plsc

NFEAT = 26
BATCH = 16384
DIM = 32
VOCAB = 100000
NC = 2
NS = 16
NW = NC * NS
BPW = BATCH // NW
NBUF = 4


def _gather_body(*refs):
    idx_refs = refs[:NFEAT]
    tab_refs = refs[NFEAT:2 * NFEAT]
    out_ref = refs[2 * NFEAT]
    rest = refs[2 * NFEAT + 1:]
    idx_all = rest[0]
    bufs = rest[1:1 + NBUF]
    gsems = rest[1 + NBUF:1 + 2 * NBUF]
    wsems = rest[1 + 2 * NBUF:1 + 3 * NBUF]
    isem = rest[1 + 3 * NBUF]
    wid = lax.axis_index("s") * NC + lax.axis_index("c")
    base = wid * BPW

    ih = [pltpu.async_copy(idx_refs[f].at[pl.ds(base, BPW)], idx_all.at[f], isem)
          for f in range(NFEAT)]
    for h in ih:
        h.wait()

    hg = [None] * NBUF
    hw = [None] * NBUF
    for f in range(NFEAT):
        s = f % NBUF
        if f >= NBUF:
            hw[s].wait()
        hg[s] = pltpu.async_copy(tab_refs[f].at[idx_all.at[f]], bufs[s], gsems[s])
        if f >= NBUF - 1:
            fp = f - (NBUF - 1)
            sp = fp % NBUF
            hg[sp].wait()
            hw[sp] = pltpu.async_copy(
                bufs[sp], out_ref.at[pl.ds(base, BPW), pl.ds(fp * DIM, DIM)],
                wsems[sp])
    for fp in range(NFEAT - (NBUF - 1), NFEAT):
        sp = fp % NBUF
        hg[sp].wait()
        hw[sp] = pltpu.async_copy(
            bufs[sp], out_ref.at[pl.ds(base, BPW), pl.ds(fp * DIM, DIM)],
            wsems[sp])
    for sp in set(fp % NBUF for fp in range(NFEAT - NBUF, NFEAT)):
        hw[sp].wait()


def kernel(f00, f01, f02, f03, f04, f05, f06, f07, f08, f09, f10, f11, f12, f13, f14, f15, f16, f17, f18, f19, f20, f21, f22, f23, f24, f25, W_f00, W_f01, W_f02, W_f03, W_f04, W_f05, W_f06, W_f07, W_f08, W_f09, W_f10, W_f11, W_f12, W_f13, W_f14, W_f15, W_f16, W_f17, W_f18, W_f19, W_f20, W_f21, W_f22, W_f23, W_f24, W_f25):
    raw_idx = (f00, f01, f02, f03, f04, f05, f06, f07, f08, f09, f10, f11,
               f12, f13, f14, f15, f16, f17, f18, f19, f20, f21, f22, f23,
               f24, f25)
    idxs = [jnp.asarray(x, jnp.int32) for x in raw_idx]
    tabs = [W_f00, W_f01, W_f02, W_f03, W_f04, W_f05, W_f06, W_f07, W_f08,
            W_f09, W_f10, W_f11, W_f12, W_f13, W_f14, W_f15, W_f16, W_f17,
            W_f18, W_f19, W_f20, W_f21, W_f22, W_f23, W_f24, W_f25]
    mesh = plsc.VectorSubcoreMesh(
        core_axis_name="c", subcore_axis_name="s", num_cores=NC, num_subcores=NS)
    run = pl.kernel(
        _gather_body,
        out_type=jax.ShapeDtypeStruct((BATCH, NFEAT * DIM), jnp.float32),
        mesh=mesh,
        compiler_params=pltpu.CompilerParams(use_tc_tiling_on_sc=False),
        scratch_types=(
            [pltpu.VMEM((NFEAT, BPW), jnp.int32)]
            + [pltpu.VMEM((BPW, DIM), jnp.float32) for _ in range(NBUF)]
            + [pltpu.SemaphoreType.DMA for _ in range(2 * NBUF + 1)]
        ),
    )
    return run(*idxs, *tabs)
